# Initial kernel scaffold; baseline (speedup 1.0000x reference)
import jax
import jax.numpy as jnp
from jax.experimental import pallas as pl


def kernel(x, edge_index, batch, W_rel, b_rel, W_root, fc1_W, fc1_b):
    raise NotImplementedError("write your pallas kernel here")


# speedup vs baseline: 5.3723x; 5.3723x over previous
import functools

import jax
import jax.numpy as jnp
from jax import lax
from jax.experimental import pallas as pl
from jax.experimental.pallas import tpu as pltpu
from jax.experimental.pallas import tpu_sc as plsc

N_NODES = 10000
D_FEAT = 128
HIDDEN = 128
N_LATENT = 128
N_EDGES = 320000
N_GRAPHS = 64

NUM_CORES = 2
NUM_SUBCORES = 16
NUM_WORKERS = NUM_CORES * NUM_SUBCORES

CHUNK = 80
EDGES_PER_WORKER = N_EDGES // NUM_WORKERS
CHUNKS_PER_WORKER = EDGES_PER_WORKER // CHUNK
ROW_CHUNK = 80
N_ROW_CHUNKS = N_NODES // ROW_CHUNK
ROW_CHUNKS_PER_SUBCORE = -(-N_ROW_CHUNKS // NUM_SUBCORES)

_sc_mesh = plsc.VectorSubcoreMesh(core_axis_name="c", subcore_axis_name="s")


@functools.partial(
    pl.kernel,
    mesh=_sc_mesh,
    out_type=jax.ShapeDtypeStruct((NUM_CORES * N_NODES, D_FEAT), jnp.float32),
    scratch_types=[
        pltpu.VMEM_SHARED((N_NODES, D_FEAT), jnp.float32),
        pltpu.VMEM((CHUNK,), jnp.int32),
        pltpu.VMEM((CHUNK,), jnp.int32),
        pltpu.VMEM((CHUNK, D_FEAT), jnp.float32),
        pltpu.SemaphoreType.DMA,
    ],
)
def _sc_aggregate(x_hbm, src_hbm, dst_hbm, zeros_hbm, out_hbm,
                  acc_sh, src_v, dst_v, rows_v, sem):
    c = lax.axis_index("c")
    s = lax.axis_index("s")

    for k in range(ROW_CHUNKS_PER_SUBCORE):
        ch = s + k * NUM_SUBCORES

        @pl.when(ch < N_ROW_CHUNKS)
        def _():
            pltpu.sync_copy(zeros_hbm, acc_sh.at[pl.ds(ch * ROW_CHUNK, ROW_CHUNK)])

    plsc.subcore_barrier()

    w = s * NUM_CORES + c
    base = w * EDGES_PER_WORKER

    def body(i, carry):
        off = base + i * CHUNK
        pltpu.sync_copy(src_hbm.at[pl.ds(off, CHUNK)], src_v)
        pltpu.sync_copy(dst_hbm.at[pl.ds(off, CHUNK)], dst_v)
        pltpu.async_copy(x_hbm.at[src_v], rows_v, sem).wait()
        pltpu.sync_copy(rows_v, acc_sh.at[dst_v], add=True)
        return carry

    lax.fori_loop(0, CHUNKS_PER_WORKER, body, 0)

    plsc.subcore_barrier()

    for k in range(ROW_CHUNKS_PER_SUBCORE):
        ch = s + k * NUM_SUBCORES

        @pl.when(ch < N_ROW_CHUNKS)
        def _():
            r0 = ch * ROW_CHUNK
            pltpu.sync_copy(acc_sh.at[pl.ds(r0, ROW_CHUNK)],
                            out_hbm.at[pl.ds(c * N_NODES + r0, ROW_CHUNK)])


_BLK = 2000
_NBLK = N_NODES // _BLK


def _tc_body(parts_ref, x_ref, batch_ref, wrel_ref, brel_ref, wroot_ref,
             fc1w_ref, fc1b_ref, out_ref, sums_ref, counts_ref):
    i = pl.program_id(0)

    @pl.when(i == 0)
    def _():
        sums_ref[...] = jnp.zeros_like(sums_ref)
        counts_ref[...] = jnp.zeros_like(counts_ref)

    agg = parts_ref[0] + parts_ref[1]
    xb = x_ref[...]
    dn_t = (((1,), (1,)), ((), ()))
    h = jnp.tanh(
        lax.dot_general(agg, wrel_ref[...], dn_t,
                        precision=lax.Precision.HIGHEST,
                        preferred_element_type=jnp.float32)
        + lax.dot_general(xb, wroot_ref[...], dn_t,
                          precision=lax.Precision.HIGHEST,
                          preferred_element_type=jnp.float32)
        + brel_ref[...]
    )

    b = batch_ref[0]
    ids = lax.broadcasted_iota(jnp.int32, (N_GRAPHS, _BLK), 0)
    onehot = (ids == b).astype(jnp.float32)
    dn = (((1,), (0,)), ((), ()))
    sums_ref[...] += lax.dot_general(onehot, h, dn,
                                     precision=lax.Precision.HIGHEST,
                                     preferred_element_type=jnp.float32)
    counts_ref[...] += lax.dot_general(onehot, jnp.ones_like(h), dn,
                                       precision=lax.Precision.HIGHEST,
                                       preferred_element_type=jnp.float32)

    @pl.when(i == pl.num_programs(0) - 1)
    def _():
        pooled = sums_ref[...] / jnp.maximum(counts_ref[...], 1.0)
        out_ref[...] = jnp.tanh(
            lax.dot_general(pooled, fc1w_ref[...], dn_t,
                            precision=lax.Precision.HIGHEST,
                            preferred_element_type=jnp.float32)
            + fc1b_ref[...]
        )


_tc_tail = pl.pallas_call(
    _tc_body,
    grid=(_NBLK,),
    in_specs=[
        pl.BlockSpec((NUM_CORES, _BLK, D_FEAT), lambda i: (0, i, 0)),
        pl.BlockSpec((_BLK, D_FEAT), lambda i: (i, 0)),
        pl.BlockSpec((1, 1, _BLK), lambda i: (i, 0, 0)),
        pl.BlockSpec((HIDDEN, D_FEAT), lambda i: (0, 0)),
        pl.BlockSpec((1, HIDDEN), lambda i: (0, 0)),
        pl.BlockSpec((HIDDEN, D_FEAT), lambda i: (0, 0)),
        pl.BlockSpec((N_LATENT, HIDDEN), lambda i: (0, 0)),
        pl.BlockSpec((1, N_LATENT), lambda i: (0, 0)),
    ],
    out_specs=pl.BlockSpec((N_GRAPHS, N_LATENT), lambda i: (0, 0)),
    out_shape=jax.ShapeDtypeStruct((N_GRAPHS, N_LATENT), jnp.float32),
    scratch_shapes=[
        pltpu.VMEM((N_GRAPHS, HIDDEN), jnp.float32),
        pltpu.VMEM((N_GRAPHS, HIDDEN), jnp.float32),
    ],
)


def kernel(x, edge_index, batch, W_rel, b_rel, W_root, fc1_W, fc1_b):
    src = edge_index[0].astype(jnp.int32)
    dst = edge_index[1].astype(jnp.int32)
    zeros = jnp.zeros((ROW_CHUNK, D_FEAT), jnp.float32)

    parts = _sc_aggregate(x, src, dst, zeros)
    parts = parts.reshape(NUM_CORES, N_NODES, D_FEAT)

    batch3 = batch.astype(jnp.int32).reshape(_NBLK, 1, _BLK)
    out = _tc_tail(parts, x, batch3, W_rel, b_rel.reshape(1, HIDDEN),
                   W_root, fc1_W, fc1_b.reshape(1, N_LATENT))
    return out
